# Initial kernel scaffold; baseline (speedup 1.0000x reference)
import functools

import jax
import jax.numpy as jnp
from jax import lax
from jax.experimental import pallas as pl
from jax.experimental.pallas import tpu as pltpu
from jax.experimental.pallas import tpu_sc as plsc

BATCH = 16384
HIST = 50
EMBED_DIM = 64
NB = BATCH * HIST
NW = 32
B_PER_W = NB // NW
CHUNK = 800
N_CHUNKS = B_PER_W // CHUNK


def _make_kernel():
  mesh = plsc.VectorSubcoreMesh(core_axis_name="c", subcore_axis_name="s")

  @functools.partial(
      pl.kernel,
      out_type=jax.ShapeDtypeStruct((NB, EMBED_DIM), jnp.float32),
      mesh=mesh,
      scratch_types=[
          pltpu.VMEM((CHUNK,), jnp.int32),
          pltpu.VMEM((CHUNK, EMBED_DIM), jnp.float32),
          pltpu.SemaphoreType.DMA,
      ],
  )
  def gather_kernel(idx_hbm, table_hbm, out_hbm, idx_v, rows_v, sem):
    wid = lax.axis_index("s") * 2 + lax.axis_index("c")
    w_base = wid * B_PER_W

    def body(ci, carry):
      base = w_base + ci * CHUNK
      pltpu.sync_copy(idx_hbm.at[pl.ds(base, CHUNK)], idx_v)
      pltpu.async_copy(table_hbm.at[idx_v], rows_v, sem).wait()
      pltpu.sync_copy(rows_v, out_hbm.at[pl.ds(base, CHUNK)])
      return carry

    lax.fori_loop(0, N_CHUNKS, body, 0)

  return gather_kernel


_gather = _make_kernel()


@jax.jit
def kernel(inputs, table):
  flat_idx = inputs.reshape(NB).astype(jnp.int32)
  out = _gather(flat_idx, table)
  return out.reshape(BATCH, HIST, EMBED_DIM)


# speedup vs baseline: 1.8302x; 1.8302x over previous
import functools

import jax
import jax.numpy as jnp
from jax import lax
from jax.experimental import pallas as pl
from jax.experimental.pallas import tpu as pltpu
from jax.experimental.pallas import tpu_sc as plsc

BATCH = 16384
HIST = 50
EMBED_DIM = 64
NB = BATCH * HIST
NW = 32
B_PER_W = NB // NW
CHUNK = 800
N_CHUNKS = B_PER_W // CHUNK


def _make_kernel():
  mesh = plsc.VectorSubcoreMesh(core_axis_name="c", subcore_axis_name="s")

  @functools.partial(
      pl.kernel,
      out_type=jax.ShapeDtypeStruct((NB, EMBED_DIM), jnp.float32),
      mesh=mesh,
      scratch_types=[
          pltpu.VMEM((CHUNK,), jnp.int32),
          pltpu.VMEM((CHUNK, EMBED_DIM), jnp.float32),
          pltpu.SemaphoreType.DMA,
      ],
      compiler_params=pltpu.CompilerParams(use_tc_tiling_on_sc=False),
  )
  def gather_kernel(idx_hbm, table_hbm, out_hbm, idx_v, rows_v, sem):
    wid = lax.axis_index("s") * 2 + lax.axis_index("c")
    w_base = wid * B_PER_W

    def body(ci, carry):
      base = w_base + ci * CHUNK
      pltpu.sync_copy(idx_hbm.at[pl.ds(base, CHUNK)], idx_v)
      pltpu.async_copy(table_hbm.at[idx_v], rows_v, sem).wait()
      pltpu.sync_copy(rows_v, out_hbm.at[pl.ds(base, CHUNK)])
      return carry

    lax.fori_loop(0, N_CHUNKS, body, 0)

  return gather_kernel


_gather = _make_kernel()


@jax.jit
def kernel(inputs, table):
  flat_idx = inputs.reshape(NB).astype(jnp.int32)
  out = _gather(flat_idx, table)
  return out.reshape(BATCH, HIST, EMBED_DIM)


# speedup vs baseline: 1.8751x; 1.0245x over previous
import functools

import jax
import jax.numpy as jnp
from jax import lax
from jax.experimental import pallas as pl
from jax.experimental.pallas import tpu as pltpu
from jax.experimental.pallas import tpu_sc as plsc

BATCH = 16384
HIST = 50
EMBED_DIM = 64
NB = BATCH * HIST
NW = 32
B_PER_W = NB // NW
CHUNK = 400
NBUF = 4
N_CHUNKS = B_PER_W // CHUNK
GROUPS = N_CHUNKS // NBUF


def _make_kernel():
  mesh = plsc.VectorSubcoreMesh(core_axis_name="c", subcore_axis_name="s")

  @functools.partial(
      pl.kernel,
      out_type=jax.ShapeDtypeStruct((NB, EMBED_DIM), jnp.float32),
      mesh=mesh,
      scratch_types=[
          pltpu.VMEM((CHUNK,), jnp.int32),
          pltpu.VMEM((CHUNK,), jnp.int32),
          pltpu.VMEM((CHUNK,), jnp.int32),
          pltpu.VMEM((CHUNK,), jnp.int32),
          pltpu.VMEM((NBUF, CHUNK, EMBED_DIM), jnp.float32),
          pltpu.SemaphoreType.DMA((NBUF,)),
          pltpu.SemaphoreType.DMA((NBUF,)),
          pltpu.SemaphoreType.DMA((NBUF,)),
      ],
      compiler_params=pltpu.CompilerParams(use_tc_tiling_on_sc=False),
  )
  def gather_kernel(idx_hbm, table_hbm, out_hbm, idx0, idx1, idx2, idx3,
                    rows_v, idx_sem, gat_sem, st_sem):
    idx_v = [idx0, idx1, idx2, idx3]
    wid = lax.axis_index("s") * 2 + lax.axis_index("c")
    w_base = wid * B_PER_W

    def issue_idx(ci, b):
      pltpu.async_copy(
          idx_hbm.at[pl.ds(w_base + ci * CHUNK, CHUNK)], idx_v[b],
          idx_sem.at[b])

    def wait_idx(ci, b):
      pltpu.make_async_copy(
          idx_hbm.at[pl.ds(w_base + ci * CHUNK, CHUNK)], idx_v[b],
          idx_sem.at[b]).wait()

    def issue_gather(b):
      pltpu.async_copy(table_hbm.at[idx_v[b]], rows_v.at[b], gat_sem.at[b])

    def wait_gather(b):
      pltpu.make_async_copy(
          table_hbm.at[idx_v[b]], rows_v.at[b], gat_sem.at[b]).wait()

    def issue_store(ci, b):
      pltpu.async_copy(
          rows_v.at[b], out_hbm.at[pl.ds(w_base + ci * CHUNK, CHUNK)],
          st_sem.at[b])

    def wait_store(ci, b):
      pltpu.make_async_copy(
          rows_v.at[b], out_hbm.at[pl.ds(w_base + ci * CHUNK, CHUNK)],
          st_sem.at[b]).wait()

    for b in range(NBUF):
      issue_idx(b, b)
    wait_idx(0, 0)
    issue_gather(0)
    wait_idx(1, 1)
    issue_gather(1)
    for b in range(NBUF):
      wait_gather(b)
      issue_store(b, b)
      issue_idx(b + NBUF, b)
      bj = (b + 2) % NBUF
      if b >= 2:
        wait_store(b - 2, bj)
      wait_idx(b + 2, bj)
      issue_gather(bj)

    def body(g, carry):
      for b in range(NBUF):
        ci = g * NBUF + b
        wait_gather(b)
        issue_store(ci, b)
        issue_idx(ci + NBUF, b)
        bj = (b + 2) % NBUF
        wait_store(ci - 2, bj)
        wait_idx(ci + 2, bj)
        issue_gather(bj)
      return carry

    lax.fori_loop(1, GROUPS - 1, body, 0)

    for b in range(NBUF):
      ci = (GROUPS - 1) * NBUF + b
      wait_gather(b)
      issue_store(ci, b)
      bj = (b + 2) % NBUF
      wait_store(ci - 2, bj)
      if ci + 2 < N_CHUNKS:
        wait_idx(ci + 2, bj)
        issue_gather(bj)
    wait_store(N_CHUNKS - 2, (N_CHUNKS - 2) % NBUF)
    wait_store(N_CHUNKS - 1, (N_CHUNKS - 1) % NBUF)

  return gather_kernel


_gather = _make_kernel()


@jax.jit
def kernel(inputs, table):
  flat_idx = inputs.reshape(NB).astype(jnp.int32)
  out = _gather(flat_idx, table)
  return out.reshape(BATCH, HIST, EMBED_DIM)


# speedup vs baseline: 1.9775x; 1.0546x over previous
import functools

import jax
import jax.numpy as jnp
from jax import lax
from jax.experimental import pallas as pl
from jax.experimental.pallas import tpu as pltpu
from jax.experimental.pallas import tpu_sc as plsc

BATCH = 16384
HIST = 50
EMBED_DIM = 64
VOCAB = 1000000
NB = BATCH * HIST
NW = 32
B_PER_W = NB // NW
CHUNK = 400
NBUF = 4
N_CHUNKS = B_PER_W // CHUNK
GROUPS = N_CHUNKS // NBUF


def _make_kernel():
  mesh = plsc.VectorSubcoreMesh(core_axis_name="c", subcore_axis_name="s")

  @functools.partial(
      pl.kernel,
      out_type=jax.ShapeDtypeStruct((NB, EMBED_DIM), jnp.float32),
      name="embed_gather",
      mesh=mesh,
      scratch_types=[
          pltpu.VMEM((CHUNK,), jnp.int32),
          pltpu.VMEM((CHUNK,), jnp.int32),
          pltpu.VMEM((CHUNK,), jnp.int32),
          pltpu.VMEM((CHUNK,), jnp.int32),
          pltpu.VMEM((NBUF, CHUNK, EMBED_DIM), jnp.float32),
          pltpu.SemaphoreType.DMA((NBUF,)),
          pltpu.SemaphoreType.DMA((NBUF,)),
          pltpu.SemaphoreType.DMA((NBUF,)),
      ],
      compiler_params=pltpu.CompilerParams(use_tc_tiling_on_sc=False),
  )
  def gather_kernel(idx_hbm, table_hbm, out_hbm, idx0, idx1, idx2, idx3,
                    rows_v, idx_sem, gat_sem, st_sem):
    idx_v = [idx0, idx1, idx2, idx3]
    wid = lax.axis_index("s") * 2 + lax.axis_index("c")
    w_base = wid * B_PER_W

    def issue_idx(ci, b):
      pltpu.async_copy(
          idx_hbm.at[pl.ds(w_base + ci * CHUNK, CHUNK)], idx_v[b],
          idx_sem.at[b])

    def wait_idx(ci, b):
      pltpu.make_async_copy(
          idx_hbm.at[pl.ds(w_base + ci * CHUNK, CHUNK)], idx_v[b],
          idx_sem.at[b]).wait()

    def issue_gather(b):
      pltpu.async_copy(table_hbm.at[idx_v[b]], rows_v.at[b], gat_sem.at[b])

    def wait_gather(b):
      pltpu.make_async_copy(
          table_hbm.at[idx_v[b]], rows_v.at[b], gat_sem.at[b]).wait()

    def issue_store(ci, b):
      pltpu.async_copy(
          rows_v.at[b], out_hbm.at[pl.ds(w_base + ci * CHUNK, CHUNK)],
          st_sem.at[b])

    def wait_store(ci, b):
      pltpu.make_async_copy(
          rows_v.at[b], out_hbm.at[pl.ds(w_base + ci * CHUNK, CHUNK)],
          st_sem.at[b]).wait()

    for b in range(NBUF):
      issue_idx(b, b)
    wait_idx(0, 0)
    issue_gather(0)
    wait_idx(1, 1)
    issue_gather(1)
    for b in range(NBUF):
      wait_gather(b)
      issue_store(b, b)
      issue_idx(b + NBUF, b)
      bj = (b + 2) % NBUF
      if b >= 2:
        wait_store(b - 2, bj)
      wait_idx(b + 2, bj)
      issue_gather(bj)

    def body(g, carry):
      for b in range(NBUF):
        ci = g * NBUF + b
        wait_gather(b)
        issue_store(ci, b)
        issue_idx(ci + NBUF, b)
        bj = (b + 2) % NBUF
        wait_store(ci - 2, bj)
        wait_idx(ci + 2, bj)
        issue_gather(bj)
      return carry

    lax.fori_loop(1, GROUPS - 1, body, 0)

    for b in range(NBUF):
      ci = (GROUPS - 1) * NBUF + b
      wait_gather(b)
      issue_store(ci, b)
      bj = (b + 2) % NBUF
      wait_store(ci - 2, bj)
      if ci + 2 < N_CHUNKS:
        wait_idx(ci + 2, bj)
        issue_gather(bj)
    wait_store(N_CHUNKS - 2, (N_CHUNKS - 2) % NBUF)
    wait_store(N_CHUNKS - 1, (N_CHUNKS - 1) % NBUF)

  return gather_kernel


_gather = _make_kernel()


@jax.jit
def kernel(inputs, table):
  tbl = jnp.pad(table, ((0, 0), (0, 128 - EMBED_DIM)))
  tbl_v = tbl.reshape(2 * VOCAB, EMBED_DIM)
  flat_idx = inputs.reshape(NB).astype(jnp.int32) * 2
  out = _gather(flat_idx, tbl_v)
  return out.reshape(BATCH, HIST, EMBED_DIM)
